# baseline (speedup 1.0000x reference)
import jax
import jax.numpy as jnp
from jax.experimental import pallas as pl

N = 10000
LAMBDA_MAX = 2.0


def _final_kernel(x_ref, t1_ref, t2_ref, w0_ref, w1_ref, w2_ref, b_ref, o_ref):
    acc = jnp.dot(x_ref[...], w0_ref[...], preferred_element_type=jnp.float32, precision=jax.lax.Precision.HIGHEST)
    acc += jnp.dot(t1_ref[...], w1_ref[...], preferred_element_type=jnp.float32, precision=jax.lax.Precision.HIGHEST)
    acc += jnp.dot(t2_ref[...], w2_ref[...], preferred_element_type=jnp.float32, precision=jax.lax.Precision.HIGHEST)
    o_ref[...] = jnp.maximum(acc + b_ref[...][None, :], 0.0)


def kernel(x, edge_index, edge_weight, W, b):
    row = edge_index[0]
    col = edge_index[1]
    mask = row != col
    ew = jnp.where(mask, edge_weight, 0.0)
    deg = jax.ops.segment_sum(ew, row, num_segments=N)
    safe_deg = jnp.where(deg > 0, deg, 1.0)
    dis = jnp.where(deg > 0, jax.lax.rsqrt(safe_deg), 0.0)
    norm = -(dis[row] * ew * dis[col]) * (2.0 / LAMBDA_MAX)

    def prop(h):
        return jax.ops.segment_sum(norm[:, None] * h[row], col, num_segments=N)

    Tx1 = prop(x)
    Tx2 = 2.0 * prop(Tx1) - x

    out = pl.pallas_call(
        _final_kernel,
        out_shape=jax.ShapeDtypeStruct((N, 128), jnp.float32),
        grid=(5,),
        in_specs=[
            pl.BlockSpec((2000, 128), lambda i: (i, 0)),
            pl.BlockSpec((2000, 128), lambda i: (i, 0)),
            pl.BlockSpec((2000, 128), lambda i: (i, 0)),
            pl.BlockSpec((128, 128), lambda i: (0, 0)),
            pl.BlockSpec((128, 128), lambda i: (0, 0)),
            pl.BlockSpec((128, 128), lambda i: (0, 0)),
            pl.BlockSpec((128,), lambda i: (0,)),
        ],
        out_specs=pl.BlockSpec((2000, 128), lambda i: (i, 0)),
    )(x, Tx1, Tx2, W[0], W[1], W[2], b)
    return out


# speedup vs baseline: 9.2394x; 9.2394x over previous
import dataclasses
import functools

import jax
import jax.numpy as jnp
from jax import lax
from jax.experimental import pallas as pl
from jax.experimental.pallas import tpu as pltpu
from jax.experimental.pallas import tpu_sc as plsc

N = 10000
E = 320000
F = 128
NPAD = 10240
LAMBDA_MAX = 2.0

NC = 2
NS = 16
NW = NC * NS
EPT = E // NW
B = 128
NB = (EPT + B - 1) // B
EPT_PAD = NB * B
F2 = F // NC
SLICES = 2 * NC
F4 = F // SLICES
EPC = E // NS
NB2 = (EPC + B - 1) // B + ((EPC + B - 1) // B) % 2
EPC_PAD = NB2 * B
ROWS_PER_TILE = NPAD // NS

_mesh = plsc.VectorSubcoreMesh(core_axis_name="c", subcore_axis_name="s")

_sc_params = pltpu.CompilerParams()
if "needs_layout_passes" in pltpu.CompilerParams.__dataclass_fields__:
    _sc_params = dataclasses.replace(_sc_params, needs_layout_passes=False)
if "use_tc_tiling_on_sc" in pltpu.CompilerParams.__dataclass_fields__:
    _sc_params = dataclasses.replace(_sc_params, use_tc_tiling_on_sc=False)


def _wid():
    return lax.axis_index("s") * NC + lax.axis_index("c")


@functools.partial(
    pl.kernel,
    out_type=jax.ShapeDtypeStruct((NW, NPAD), jnp.float32),
    mesh=_mesh,
    scratch_types=[
        pltpu.VMEM((EPT_PAD,), jnp.int32),
        pltpu.VMEM((EPT_PAD,), jnp.int32),
        pltpu.VMEM((EPT_PAD,), jnp.float32),
        pltpu.VMEM((NPAD,), jnp.float32),
    ],
)
def _deg_kernel(row_hbm, col_hbm, ew_hbm, out_hbm, row_v, col_v, ew_v, deg_v):
    w = _wid()
    pltpu.sync_copy(row_hbm.at[w], row_v)
    pltpu.sync_copy(col_hbm.at[w], col_v)
    pltpu.sync_copy(ew_hbm.at[w], ew_v)

    zeros16 = jnp.zeros((16,), jnp.float32)

    @pl.loop(0, NPAD, step=16)
    def _(i):
        deg_v[pl.ds(i, 16)] = zeros16

    lane0 = lax.iota(jnp.int32, 16) == 0

    @pl.loop(0, EPT_PAD, step=16)
    def _(i):
        rv = row_v[pl.ds(i, 16)]
        cv = col_v[pl.ds(i, 16)]
        wv = ew_v[pl.ds(i, 16)]
        wm = jnp.where(rv != cv, wv, 0.0)
        for l in range(16):
            r = rv[l]
            d = deg_v[pl.ds(r, 16)]
            deg_v[pl.ds(r, 16)] = d + jnp.where(lane0, wm[l], 0.0)

    pltpu.sync_copy(deg_v, out_hbm.at[w])


def _dis_body(parts_ref, o_ref):
    deg = jnp.sum(parts_ref[...], axis=0, keepdims=True)
    safe = jnp.where(deg > 0, deg, 1.0)
    o_ref[...] = jnp.where(deg > 0, lax.rsqrt(safe), 0.0)


def _dis_tc(parts):
    return pl.pallas_call(
        _dis_body,
        out_shape=jax.ShapeDtypeStruct((1, NPAD), jnp.float32),
    )(parts)


def _prop_body(compute_norm, *refs):
    if compute_norm:
        (row_hbm, col_hbm, ew_hbm, dis_hbm, h_hbm,
         acc_hbm, norm_hbm,
         row_v, col_v, aux_v, dis_v, norm_v, g0, g1, acc_sh, sem0, sem1) = refs
    else:
        (row_hbm, col_hbm, norm_src_hbm, h_hbm,
         acc_hbm,
         row_v, col_v, norm_v, g0, g1, acc_sh, sem0, sem1) = refs

    core = lax.axis_index("c")
    sid = lax.axis_index("s")

    pltpu.sync_copy(row_hbm.at[sid], row_v)
    pltpu.sync_copy(col_hbm.at[sid], col_v)

    if compute_norm:
        pltpu.sync_copy(ew_hbm.at[sid], aux_v)
        pltpu.sync_copy(dis_hbm, dis_v)
        scale = jnp.float32(2.0 / LAMBDA_MAX)

        @pl.loop(0, NB2)
        def _(b):
            @pl.loop(0, B, step=16)
            def _(j):
                ir = row_v[b, pl.ds(j, 16)]
                ic = col_v[b, pl.ds(j, 16)]
                dr = plsc.load_gather(dis_v, [ir])
                dc = plsc.load_gather(dis_v, [ic])
                ww = aux_v[b, pl.ds(j, 16)]
                nrm = -(dr * ww * dc) * scale
                norm_v[b, pl.ds(j, 16)] = jnp.where(ir != ic, nrm, 0.0)

        @pl.when(core == 0)
        def _():
            pltpu.sync_copy(norm_v, norm_hbm.at[sid])
    else:
        pltpu.sync_copy(norm_src_hbm.at[sid], norm_v)

    zeros16 = jnp.zeros((16,), jnp.float32)

    def scale_rows(gbuf, b):
        @pl.loop(0, B, step=16)
        def _(j):
            nv = norm_v[b, pl.ds(j, 16)]
            for l in range(16):
                s = nv[l]
                for k in range(F4 // 16):
                    slc = pl.ds(k * 16, 16)
                    gbuf[j + l, slc] = gbuf[j + l, slc] * s

    for sub in range(2):
        sl = core * 2 + sub

        @pl.loop(0, B)
        def _(j):
            for k in range(F4 // 16):
                g0[j, pl.ds(k * 16, 16)] = zeros16

        for t in range(ROWS_PER_TILE // B):
            pltpu.sync_copy(g0, acc_sh.at[pl.ds(sid * ROWS_PER_TILE + t * B, B)])
        plsc.subcore_barrier()

        def gather(gbuf, sem, b):
            return pltpu.async_copy(h_hbm.at[sl].at[row_v.at[b]], gbuf, sem)

        gather(g0, sem0, 0)

        @pl.loop(0, NB2, step=2)
        def _(b):
            pltpu.make_async_copy(h_hbm.at[sl].at[row_v.at[b]], g0, sem0).wait()
            gather(g1, sem1, b + 1)
            scale_rows(g0, b)
            pltpu.sync_copy(g0, acc_sh.at[col_v.at[b]], add=True)

            pltpu.make_async_copy(h_hbm.at[sl].at[row_v.at[b + 1]], g1, sem1).wait()

            @pl.when(b + 2 < NB2)
            def _():
                gather(g0, sem0, b + 2)

            scale_rows(g1, b + 1)
            pltpu.sync_copy(g1, acc_sh.at[col_v.at[b + 1]], add=True)

        plsc.subcore_barrier()
        pltpu.sync_copy(
            acc_sh.at[pl.ds(sid * ROWS_PER_TILE, ROWS_PER_TILE)],
            acc_hbm.at[sl, pl.ds(sid * ROWS_PER_TILE, ROWS_PER_TILE)],
        )


_ACC_TYPE = jax.ShapeDtypeStruct((SLICES, NPAD, F4), jnp.float32)
_NORM_TYPE = jax.ShapeDtypeStruct((NS, NB2, B), jnp.float32)

_prop1 = pl.kernel(
    functools.partial(_prop_body, True),
    out_type=(_ACC_TYPE, _NORM_TYPE),
    mesh=_mesh,
    compiler_params=_sc_params,
    scratch_types=[
        pltpu.VMEM((NB2, B), jnp.int32),
        pltpu.VMEM((NB2, B), jnp.int32),
        pltpu.VMEM((NB2, B), jnp.float32),
        pltpu.VMEM((NPAD,), jnp.float32),
        pltpu.VMEM((NB2, B), jnp.float32),
        pltpu.VMEM((B, F4), jnp.float32),
        pltpu.VMEM((B, F4), jnp.float32),
        pltpu.VMEM_SHARED((NPAD, F4), jnp.float32),
        pltpu.SemaphoreType.DMA,
        pltpu.SemaphoreType.DMA,
    ],
)

_prop2 = pl.kernel(
    functools.partial(_prop_body, False),
    out_type=_ACC_TYPE,
    mesh=_mesh,
    compiler_params=_sc_params,
    scratch_types=[
        pltpu.VMEM((NB2, B), jnp.int32),
        pltpu.VMEM((NB2, B), jnp.int32),
        pltpu.VMEM((NB2, B), jnp.float32),
        pltpu.VMEM((B, F4), jnp.float32),
        pltpu.VMEM((B, F4), jnp.float32),
        pltpu.VMEM_SHARED((NPAD, F4), jnp.float32),
        pltpu.SemaphoreType.DMA,
        pltpu.SemaphoreType.DMA,
    ],
)


def _final_body(x_ref, p1_ref, p2_ref, w0_ref, w1_ref, w2_ref, b_ref, o_ref):
    dot = functools.partial(
        jnp.dot,
        preferred_element_type=jnp.float32,
        precision=lax.Precision.HIGHEST,
    )
    x = x_ref[...]
    y1 = jnp.concatenate([p1_ref[s] for s in range(SLICES)], axis=1)
    y2 = jnp.concatenate([p2_ref[s] for s in range(SLICES)], axis=1)
    t2 = 2.0 * y2 - x
    acc = dot(x, w0_ref[...]) + dot(y1, w1_ref[...]) + dot(t2, w2_ref[...])
    o_ref[...] = jnp.maximum(acc + b_ref[...][None, :], 0.0)


def _final_tc(x, p1, p2, W, b):
    blk = 2000
    return pl.pallas_call(
        _final_body,
        out_shape=jax.ShapeDtypeStruct((N, F), jnp.float32),
        grid=(N // blk,),
        in_specs=[
            pl.BlockSpec((blk, F), lambda i: (i, 0)),
            pl.BlockSpec((SLICES, blk, F4), lambda i: (0, i, 0)),
            pl.BlockSpec((SLICES, blk, F4), lambda i: (0, i, 0)),
            pl.BlockSpec((F, F), lambda i: (0, 0)),
            pl.BlockSpec((F, F), lambda i: (0, 0)),
            pl.BlockSpec((F, F), lambda i: (0, 0)),
            pl.BlockSpec((F,), lambda i: (0,)),
        ],
        out_specs=pl.BlockSpec((blk, F), lambda i: (i, 0)),
    )(x, p1, p2, W[0], W[1], W[2], b)


def kernel(x, edge_index, edge_weight, W, b):
    row = edge_index[0]
    col = edge_index[1]

    row_p = jnp.pad(row.reshape(NW, EPT), ((0, 0), (0, EPT_PAD - EPT)))
    col_p = jnp.pad(col.reshape(NW, EPT), ((0, 0), (0, EPT_PAD - EPT)))
    ew_p = jnp.pad(edge_weight.reshape(NW, EPT), ((0, 0), (0, EPT_PAD - EPT)))

    row3 = jnp.pad(row.reshape(NS, EPC), ((0, 0), (0, EPC_PAD - EPC))).reshape(NS, NB2, B)
    col3 = jnp.pad(col.reshape(NS, EPC), ((0, 0), (0, EPC_PAD - EPC))).reshape(NS, NB2, B)
    ew3 = jnp.pad(edge_weight.reshape(NS, EPC), ((0, 0), (0, EPC_PAD - EPC))).reshape(NS, NB2, B)

    x_pad = jnp.pad(x, ((0, NPAD - N), (0, 0)))
    x_split = jnp.stack([x_pad[:, s * F4:(s + 1) * F4] for s in range(SLICES)])

    deg_parts = _deg_kernel(row_p, col_p, ew_p)
    dis = _dis_tc(deg_parts).reshape(NPAD)

    p1, norm3 = _prop1(row3, col3, ew3, dis, x_split)
    p2 = _prop2(row3, col3, norm3, p1)

    return _final_tc(x, p1, p2, W, b)


# speedup vs baseline: 11.7125x; 1.2677x over previous
import dataclasses
import functools

import jax
import jax.numpy as jnp
from jax import lax
from jax.experimental import pallas as pl
from jax.experimental.pallas import tpu as pltpu
from jax.experimental.pallas import tpu_sc as plsc

N = 10000
E = 320000
F = 128
NPAD = 10240
LAMBDA_MAX = 2.0

NC = 2
NS = 16
NW = NC * NS
EPT = E // NW
B = 128
NB = (EPT + B - 1) // B
EPT_PAD = NB * B
F2 = F // NC
SLICES = 2 * NC
F4 = F // SLICES
EPC = E // NS
NB2 = (EPC + B - 1) // B + ((EPC + B - 1) // B) % 2
EPC_PAD = NB2 * B
ROWS_PER_TILE = NPAD // NS

_mesh = plsc.VectorSubcoreMesh(core_axis_name="c", subcore_axis_name="s")

_sc_params = pltpu.CompilerParams()
if "needs_layout_passes" in pltpu.CompilerParams.__dataclass_fields__:
    _sc_params = dataclasses.replace(_sc_params, needs_layout_passes=False)
if "use_tc_tiling_on_sc" in pltpu.CompilerParams.__dataclass_fields__:
    _sc_params = dataclasses.replace(_sc_params, use_tc_tiling_on_sc=False)


def _wid():
    return lax.axis_index("s") * NC + lax.axis_index("c")


@functools.partial(
    pl.kernel,
    out_type=jax.ShapeDtypeStruct((NW, NPAD), jnp.float32),
    mesh=_mesh,
    scratch_types=[
        pltpu.VMEM((EPT_PAD,), jnp.int32),
        pltpu.VMEM((EPT_PAD,), jnp.int32),
        pltpu.VMEM((EPT_PAD,), jnp.float32),
        pltpu.VMEM((NPAD,), jnp.float32),
    ],
)
def _deg_kernel(row_hbm, col_hbm, ew_hbm, out_hbm, row_v, col_v, ew_v, deg_v):
    w = _wid()
    pltpu.sync_copy(row_hbm.at[w], row_v)
    pltpu.sync_copy(col_hbm.at[w], col_v)
    pltpu.sync_copy(ew_hbm.at[w], ew_v)

    zeros16 = jnp.zeros((16,), jnp.float32)

    @pl.loop(0, NPAD, step=16)
    def _(i):
        deg_v[pl.ds(i, 16)] = zeros16

    lane0 = lax.iota(jnp.int32, 16) == 0

    @pl.loop(0, EPT_PAD, step=16)
    def _(i):
        rv = row_v[pl.ds(i, 16)]
        cv = col_v[pl.ds(i, 16)]
        wv = ew_v[pl.ds(i, 16)]
        wm = jnp.where(rv != cv, wv, 0.0)
        for l in range(16):
            r = rv[l]
            d = deg_v[pl.ds(r, 16)]
            deg_v[pl.ds(r, 16)] = d + jnp.where(lane0, wm[l], 0.0)

    pltpu.sync_copy(deg_v, out_hbm.at[w])


def _dis_body(parts_ref, o_ref):
    deg = jnp.sum(parts_ref[...], axis=0, keepdims=True)
    safe = jnp.where(deg > 0, deg, 1.0)
    o_ref[...] = jnp.where(deg > 0, lax.rsqrt(safe), 0.0)


def _dis_tc(parts):
    return pl.pallas_call(
        _dis_body,
        out_shape=jax.ShapeDtypeStruct((1, NPAD), jnp.float32),
    )(parts)


def _prop_body(compute_norm, *refs):
    if compute_norm:
        (row_hbm, col_hbm, ew_hbm, dis_hbm, h_hbm,
         acc_hbm, norm_hbm,
         row_v, col_v, aux_v, dis_v, norm_v, g0, g1, s0, s1,
         acc_sh, sem0, sem1, ssem0, ssem1) = refs
    else:
        (row_hbm, col_hbm, norm_src_hbm, h_hbm,
         acc_hbm,
         row_v, col_v, norm_v, g0, g1, s0, s1,
         acc_sh, sem0, sem1, ssem0, ssem1) = refs

    core = lax.axis_index("c")
    sid = lax.axis_index("s")

    pltpu.sync_copy(row_hbm.at[sid], row_v)
    pltpu.sync_copy(col_hbm.at[sid], col_v)

    if compute_norm:
        pltpu.sync_copy(ew_hbm.at[sid], aux_v)
        pltpu.sync_copy(dis_hbm, dis_v)
        scale = jnp.float32(2.0 / LAMBDA_MAX)

        @pl.loop(0, NB2)
        def _(b):
            @pl.loop(0, B, step=16)
            def _(j):
                ir = row_v[b, pl.ds(j, 16)]
                ic = col_v[b, pl.ds(j, 16)]
                dr = plsc.load_gather(dis_v, [ir])
                dc = plsc.load_gather(dis_v, [ic])
                ww = aux_v[b, pl.ds(j, 16)]
                nrm = -(dr * ww * dc) * scale
                norm_v[b, pl.ds(j, 16)] = jnp.where(ir != ic, nrm, 0.0)

        @pl.when(core == 0)
        def _():
            pltpu.sync_copy(norm_v, norm_hbm.at[sid])
    else:
        pltpu.sync_copy(norm_src_hbm.at[sid], norm_v)

    zeros16 = jnp.zeros((16,), jnp.float32)

    def scale_rows(gbuf, sbuf, b):
        @pl.loop(0, B, step=16)
        def _(j):
            nv = norm_v[b, pl.ds(j, 16)]
            for l in range(16):
                s = nv[l]
                for k in range(F4 // 16):
                    slc = pl.ds(k * 16, 16)
                    sbuf[j + l, slc] = gbuf[j + l, slc] * s

    for sub in range(2):
        sl = core * 2 + sub

        @pl.loop(0, B)
        def _(j):
            for k in range(F4 // 16):
                g0[j, pl.ds(k * 16, 16)] = zeros16

        for t in range(ROWS_PER_TILE // B):
            pltpu.sync_copy(g0, acc_sh.at[pl.ds(sid * ROWS_PER_TILE + t * B, B)])
        plsc.subcore_barrier()

        def gather(gbuf, sem, b):
            return pltpu.async_copy(h_hbm.at[sl].at[row_v.at[b]], gbuf, sem)

        def wait_gather(gbuf, sem, b):
            pltpu.make_async_copy(h_hbm.at[sl].at[row_v.at[b]], gbuf, sem).wait()

        def scatter(sbuf, sem, b):
            return pltpu.async_copy(sbuf, acc_sh.at[col_v.at[b]], sem, add=True)

        def wait_scatter(sbuf, sem, b):
            pltpu.make_async_copy(sbuf, acc_sh.at[col_v.at[b]], sem).wait()

        gather(g0, sem0, 0)
        gather(g1, sem1, 1)

        @pl.loop(0, NB2, step=2)
        def _(b):
            wait_gather(g0, sem0, b)

            @pl.when(b >= 2)
            def _():
                wait_scatter(s0, ssem0, b - 2)

            scale_rows(g0, s0, b)
            scatter(s0, ssem0, b)

            @pl.when(b + 2 < NB2)
            def _():
                gather(g0, sem0, b + 2)

            wait_gather(g1, sem1, b + 1)

            @pl.when(b >= 2)
            def _():
                wait_scatter(s1, ssem1, b - 1)

            scale_rows(g1, s1, b + 1)
            scatter(s1, ssem1, b + 1)

            @pl.when(b + 3 < NB2)
            def _():
                gather(g1, sem1, b + 3)

        wait_scatter(s0, ssem0, NB2 - 2)
        wait_scatter(s1, ssem1, NB2 - 1)

        plsc.subcore_barrier()
        pltpu.sync_copy(
            acc_sh.at[pl.ds(sid * ROWS_PER_TILE, ROWS_PER_TILE)],
            acc_hbm.at[sl, pl.ds(sid * ROWS_PER_TILE, ROWS_PER_TILE)],
        )


_ACC_TYPE = jax.ShapeDtypeStruct((SLICES, NPAD, F4), jnp.float32)
_NORM_TYPE = jax.ShapeDtypeStruct((NS, NB2, B), jnp.float32)

_prop1 = pl.kernel(
    functools.partial(_prop_body, True),
    out_type=(_ACC_TYPE, _NORM_TYPE),
    mesh=_mesh,
    compiler_params=_sc_params,
    scratch_types=[
        pltpu.VMEM((NB2, B), jnp.int32),
        pltpu.VMEM((NB2, B), jnp.int32),
        pltpu.VMEM((NB2, B), jnp.float32),
        pltpu.VMEM((NPAD,), jnp.float32),
        pltpu.VMEM((NB2, B), jnp.float32),
        pltpu.VMEM((B, F4), jnp.float32),
        pltpu.VMEM((B, F4), jnp.float32),
        pltpu.VMEM((B, F4), jnp.float32),
        pltpu.VMEM((B, F4), jnp.float32),
        pltpu.VMEM_SHARED((NPAD, F4), jnp.float32),
        pltpu.SemaphoreType.DMA,
        pltpu.SemaphoreType.DMA,
        pltpu.SemaphoreType.DMA,
        pltpu.SemaphoreType.DMA,
    ],
)

_prop2 = pl.kernel(
    functools.partial(_prop_body, False),
    out_type=_ACC_TYPE,
    mesh=_mesh,
    compiler_params=_sc_params,
    scratch_types=[
        pltpu.VMEM((NB2, B), jnp.int32),
        pltpu.VMEM((NB2, B), jnp.int32),
        pltpu.VMEM((NB2, B), jnp.float32),
        pltpu.VMEM((B, F4), jnp.float32),
        pltpu.VMEM((B, F4), jnp.float32),
        pltpu.VMEM((B, F4), jnp.float32),
        pltpu.VMEM((B, F4), jnp.float32),
        pltpu.VMEM_SHARED((NPAD, F4), jnp.float32),
        pltpu.SemaphoreType.DMA,
        pltpu.SemaphoreType.DMA,
        pltpu.SemaphoreType.DMA,
        pltpu.SemaphoreType.DMA,
    ],
)


def _final_body(x_ref, p1_ref, p2_ref, w0_ref, w1_ref, w2_ref, b_ref, o_ref):
    dot = functools.partial(
        jnp.dot,
        preferred_element_type=jnp.float32,
        precision=lax.Precision.HIGHEST,
    )
    x = x_ref[...]
    y1 = jnp.concatenate([p1_ref[s] for s in range(SLICES)], axis=1)
    y2 = jnp.concatenate([p2_ref[s] for s in range(SLICES)], axis=1)
    t2 = 2.0 * y2 - x
    acc = dot(x, w0_ref[...]) + dot(y1, w1_ref[...]) + dot(t2, w2_ref[...])
    o_ref[...] = jnp.maximum(acc + b_ref[...][None, :], 0.0)


def _final_tc(x, p1, p2, W, b):
    blk = 2000
    return pl.pallas_call(
        _final_body,
        out_shape=jax.ShapeDtypeStruct((N, F), jnp.float32),
        grid=(N // blk,),
        in_specs=[
            pl.BlockSpec((blk, F), lambda i: (i, 0)),
            pl.BlockSpec((SLICES, blk, F4), lambda i: (0, i, 0)),
            pl.BlockSpec((SLICES, blk, F4), lambda i: (0, i, 0)),
            pl.BlockSpec((F, F), lambda i: (0, 0)),
            pl.BlockSpec((F, F), lambda i: (0, 0)),
            pl.BlockSpec((F, F), lambda i: (0, 0)),
            pl.BlockSpec((F,), lambda i: (0,)),
        ],
        out_specs=pl.BlockSpec((blk, F), lambda i: (i, 0)),
    )(x, p1, p2, W[0], W[1], W[2], b)


def kernel(x, edge_index, edge_weight, W, b):
    row = edge_index[0]
    col = edge_index[1]

    row_p = jnp.pad(row.reshape(NW, EPT), ((0, 0), (0, EPT_PAD - EPT)))
    col_p = jnp.pad(col.reshape(NW, EPT), ((0, 0), (0, EPT_PAD - EPT)))
    ew_p = jnp.pad(edge_weight.reshape(NW, EPT), ((0, 0), (0, EPT_PAD - EPT)))

    row3 = jnp.pad(row.reshape(NS, EPC), ((0, 0), (0, EPC_PAD - EPC))).reshape(NS, NB2, B)
    col3 = jnp.pad(col.reshape(NS, EPC), ((0, 0), (0, EPC_PAD - EPC))).reshape(NS, NB2, B)
    ew3 = jnp.pad(edge_weight.reshape(NS, EPC), ((0, 0), (0, EPC_PAD - EPC))).reshape(NS, NB2, B)

    x_pad = jnp.pad(x, ((0, NPAD - N), (0, 0)))
    x_split = jnp.stack([x_pad[:, s * F4:(s + 1) * F4] for s in range(SLICES)])

    deg_parts = _deg_kernel(row_p, col_p, ew_p)
    dis = _dis_tc(deg_parts).reshape(NPAD)

    p1, norm3 = _prop1(row3, col3, ew3, dis, x_split)
    p2 = _prop2(row3, col3, norm3, p1)

    return _final_tc(x, p1, p2, W, b)


# speedup vs baseline: 12.1872x; 1.0405x over previous
import dataclasses
import functools

import jax
import jax.numpy as jnp
from jax import lax
from jax.experimental import pallas as pl
from jax.experimental.pallas import tpu as pltpu
from jax.experimental.pallas import tpu_sc as plsc

N = 10000
E = 320000
F = 128
NPAD = 10240
LAMBDA_MAX = 2.0

NC = 2
NS = 16
NW = NC * NS
EPT = E // NW
B = 128
NB = (EPT + B - 1) // B
EPT_PAD = NB * B
F2 = F // NC
SLICES = 2 * NC
F4 = F // SLICES
EPC = E // NS
NB2 = (EPC + B - 1) // B + ((EPC + B - 1) // B) % 2
EPC_PAD = NB2 * B
ROWS_PER_TILE = NPAD // NS

_mesh = plsc.VectorSubcoreMesh(core_axis_name="c", subcore_axis_name="s")

_sc_params = pltpu.CompilerParams()
if "needs_layout_passes" in pltpu.CompilerParams.__dataclass_fields__:
    _sc_params = dataclasses.replace(_sc_params, needs_layout_passes=False)
if "use_tc_tiling_on_sc" in pltpu.CompilerParams.__dataclass_fields__:
    _sc_params = dataclasses.replace(_sc_params, use_tc_tiling_on_sc=False)


def _wid():
    return lax.axis_index("s") * NC + lax.axis_index("c")


@functools.partial(
    pl.kernel,
    out_type=jax.ShapeDtypeStruct((NW, NPAD), jnp.float32),
    mesh=_mesh,
    compiler_params=_sc_params,
    scratch_types=[
        pltpu.VMEM((EPT_PAD,), jnp.int32),
        pltpu.VMEM((EPT_PAD,), jnp.int32),
        pltpu.VMEM((EPT_PAD,), jnp.float32),
        pltpu.VMEM((NPAD,), jnp.float32),
    ],
)
def _deg_kernel(row_hbm, col_hbm, ew_hbm, out_hbm, row_v, col_v, ew_v, deg_v):
    w = _wid()
    pltpu.sync_copy(row_hbm.at[w], row_v)
    pltpu.sync_copy(col_hbm.at[w], col_v)
    pltpu.sync_copy(ew_hbm.at[w], ew_v)

    zeros16 = jnp.zeros((16,), jnp.float32)

    @pl.loop(0, NPAD, step=16)
    def _(i):
        deg_v[pl.ds(i, 16)] = zeros16

    @pl.loop(0, EPT_PAD, step=16)
    def _(i):
        rv = row_v[pl.ds(i, 16)]
        cv = col_v[pl.ds(i, 16)]
        wv = ew_v[pl.ds(i, 16)]
        wm = jnp.where(rv != cv, wv, 0.0)
        plsc.addupdate_scatter(deg_v, [rv], wm)

    pltpu.sync_copy(deg_v, out_hbm.at[w])


def _dis_body(parts_ref, o_ref):
    deg = jnp.sum(parts_ref[...], axis=0, keepdims=True)
    safe = jnp.where(deg > 0, deg, 1.0)
    o_ref[...] = jnp.where(deg > 0, lax.rsqrt(safe), 0.0)


def _dis_tc(parts):
    return pl.pallas_call(
        _dis_body,
        out_shape=jax.ShapeDtypeStruct((1, NPAD), jnp.float32),
    )(parts)


def _prop_body(compute_norm, *refs):
    if compute_norm:
        (row_hbm, col_hbm, ew_hbm, dis_hbm, h_hbm,
         acc_hbm, norm_hbm,
         row_v, col_v, aux_v, dis_v, norm_v, g0, g1, s0, s1,
         acc_sh, sem0, sem1, ssem0, ssem1) = refs
    else:
        (row_hbm, col_hbm, norm_src_hbm, h_hbm,
         acc_hbm,
         row_v, col_v, norm_v, g0, g1, s0, s1,
         acc_sh, sem0, sem1, ssem0, ssem1) = refs

    core = lax.axis_index("c")
    sid = lax.axis_index("s")

    pltpu.sync_copy(row_hbm.at[sid], row_v)
    pltpu.sync_copy(col_hbm.at[sid], col_v)

    if compute_norm:
        pltpu.sync_copy(ew_hbm.at[sid], aux_v)
        pltpu.sync_copy(dis_hbm, dis_v)
        scale = jnp.float32(2.0 / LAMBDA_MAX)

        @pl.loop(0, NB2)
        def _(b):
            @pl.loop(0, B, step=16)
            def _(j):
                ir = row_v[b, pl.ds(j, 16)]
                ic = col_v[b, pl.ds(j, 16)]
                dr = plsc.load_gather(dis_v, [ir])
                dc = plsc.load_gather(dis_v, [ic])
                ww = aux_v[b, pl.ds(j, 16)]
                nrm = -(dr * ww * dc) * scale
                norm_v[b, pl.ds(j, 16)] = jnp.where(ir != ic, nrm, 0.0)

        @pl.when(core == 0)
        def _():
            pltpu.sync_copy(norm_v, norm_hbm.at[sid])
    else:
        pltpu.sync_copy(norm_src_hbm.at[sid], norm_v)

    zeros16 = jnp.zeros((16,), jnp.float32)

    def scale_rows(gbuf, sbuf, b):
        @pl.loop(0, B, step=16)
        def _(j):
            nv = norm_v[b, pl.ds(j, 16)]
            for l in range(16):
                s = nv[l]
                for k in range(F4 // 16):
                    slc = pl.ds(k * 16, 16)
                    sbuf[j + l, slc] = gbuf[j + l, slc] * s

    for sub in range(2):
        sl = core * 2 + sub

        @pl.loop(0, B)
        def _(j):
            for k in range(F4 // 16):
                g0[j, pl.ds(k * 16, 16)] = zeros16

        for t in range(ROWS_PER_TILE // B):
            pltpu.sync_copy(g0, acc_sh.at[pl.ds(sid * ROWS_PER_TILE + t * B, B)])
        plsc.subcore_barrier()

        def gather(gbuf, sem, b):
            return pltpu.async_copy(h_hbm.at[sl].at[row_v.at[b]], gbuf, sem)

        def wait_gather(gbuf, sem, b):
            pltpu.make_async_copy(h_hbm.at[sl].at[row_v.at[b]], gbuf, sem).wait()

        def scatter(sbuf, sem, b):
            return pltpu.async_copy(sbuf, acc_sh.at[col_v.at[b]], sem, add=True)

        def wait_scatter(sbuf, sem, b):
            pltpu.make_async_copy(sbuf, acc_sh.at[col_v.at[b]], sem).wait()

        gather(g0, sem0, 0)
        gather(g1, sem1, 1)

        @pl.loop(0, NB2, step=2)
        def _(b):
            wait_gather(g0, sem0, b)

            @pl.when(b >= 2)
            def _():
                wait_scatter(s0, ssem0, b - 2)

            scale_rows(g0, s0, b)
            scatter(s0, ssem0, b)

            @pl.when(b + 2 < NB2)
            def _():
                gather(g0, sem0, b + 2)

            wait_gather(g1, sem1, b + 1)

            @pl.when(b >= 2)
            def _():
                wait_scatter(s1, ssem1, b - 1)

            scale_rows(g1, s1, b + 1)
            scatter(s1, ssem1, b + 1)

            @pl.when(b + 3 < NB2)
            def _():
                gather(g1, sem1, b + 3)

        wait_scatter(s0, ssem0, NB2 - 2)
        wait_scatter(s1, ssem1, NB2 - 1)

        plsc.subcore_barrier()
        pltpu.sync_copy(
            acc_sh.at[pl.ds(sid * ROWS_PER_TILE, ROWS_PER_TILE)],
            acc_hbm.at[sl, pl.ds(sid * ROWS_PER_TILE, ROWS_PER_TILE)],
        )


_ACC_TYPE = jax.ShapeDtypeStruct((SLICES, NPAD, F4), jnp.float32)
_NORM_TYPE = jax.ShapeDtypeStruct((NS, NB2, B), jnp.float32)

_prop1 = pl.kernel(
    functools.partial(_prop_body, True),
    out_type=(_ACC_TYPE, _NORM_TYPE),
    mesh=_mesh,
    compiler_params=_sc_params,
    scratch_types=[
        pltpu.VMEM((NB2, B), jnp.int32),
        pltpu.VMEM((NB2, B), jnp.int32),
        pltpu.VMEM((NB2, B), jnp.float32),
        pltpu.VMEM((NPAD,), jnp.float32),
        pltpu.VMEM((NB2, B), jnp.float32),
        pltpu.VMEM((B, F4), jnp.float32),
        pltpu.VMEM((B, F4), jnp.float32),
        pltpu.VMEM((B, F4), jnp.float32),
        pltpu.VMEM((B, F4), jnp.float32),
        pltpu.VMEM_SHARED((NPAD, F4), jnp.float32),
        pltpu.SemaphoreType.DMA,
        pltpu.SemaphoreType.DMA,
        pltpu.SemaphoreType.DMA,
        pltpu.SemaphoreType.DMA,
    ],
)

_prop2 = pl.kernel(
    functools.partial(_prop_body, False),
    out_type=_ACC_TYPE,
    mesh=_mesh,
    compiler_params=_sc_params,
    scratch_types=[
        pltpu.VMEM((NB2, B), jnp.int32),
        pltpu.VMEM((NB2, B), jnp.int32),
        pltpu.VMEM((NB2, B), jnp.float32),
        pltpu.VMEM((B, F4), jnp.float32),
        pltpu.VMEM((B, F4), jnp.float32),
        pltpu.VMEM((B, F4), jnp.float32),
        pltpu.VMEM((B, F4), jnp.float32),
        pltpu.VMEM_SHARED((NPAD, F4), jnp.float32),
        pltpu.SemaphoreType.DMA,
        pltpu.SemaphoreType.DMA,
        pltpu.SemaphoreType.DMA,
        pltpu.SemaphoreType.DMA,
    ],
)


def _final_body(x_ref, p1_ref, p2_ref, w0_ref, w1_ref, w2_ref, b_ref, o_ref):
    dot = functools.partial(
        jnp.dot,
        preferred_element_type=jnp.float32,
        precision=lax.Precision.HIGHEST,
    )
    x = x_ref[...]
    y1 = jnp.concatenate([p1_ref[s] for s in range(SLICES)], axis=1)
    y2 = jnp.concatenate([p2_ref[s] for s in range(SLICES)], axis=1)
    t2 = 2.0 * y2 - x
    acc = dot(x, w0_ref[...]) + dot(y1, w1_ref[...]) + dot(t2, w2_ref[...])
    o_ref[...] = jnp.maximum(acc + b_ref[...][None, :], 0.0)


def _final_tc(x, p1, p2, W, b):
    blk = 2000
    return pl.pallas_call(
        _final_body,
        out_shape=jax.ShapeDtypeStruct((N, F), jnp.float32),
        grid=(N // blk,),
        in_specs=[
            pl.BlockSpec((blk, F), lambda i: (i, 0)),
            pl.BlockSpec((SLICES, blk, F4), lambda i: (0, i, 0)),
            pl.BlockSpec((SLICES, blk, F4), lambda i: (0, i, 0)),
            pl.BlockSpec((F, F), lambda i: (0, 0)),
            pl.BlockSpec((F, F), lambda i: (0, 0)),
            pl.BlockSpec((F, F), lambda i: (0, 0)),
            pl.BlockSpec((F,), lambda i: (0,)),
        ],
        out_specs=pl.BlockSpec((blk, F), lambda i: (i, 0)),
    )(x, p1, p2, W[0], W[1], W[2], b)


def kernel(x, edge_index, edge_weight, W, b):
    row = edge_index[0]
    col = edge_index[1]

    row_p = jnp.pad(row.reshape(NW, EPT), ((0, 0), (0, EPT_PAD - EPT)))
    col_p = jnp.pad(col.reshape(NW, EPT), ((0, 0), (0, EPT_PAD - EPT)))
    ew_p = jnp.pad(edge_weight.reshape(NW, EPT), ((0, 0), (0, EPT_PAD - EPT)))

    row3 = jnp.pad(row.reshape(NS, EPC), ((0, 0), (0, EPC_PAD - EPC))).reshape(NS, NB2, B)
    col3 = jnp.pad(col.reshape(NS, EPC), ((0, 0), (0, EPC_PAD - EPC))).reshape(NS, NB2, B)
    ew3 = jnp.pad(edge_weight.reshape(NS, EPC), ((0, 0), (0, EPC_PAD - EPC))).reshape(NS, NB2, B)

    x_pad = jnp.pad(x, ((0, NPAD - N), (0, 0)))
    x_split = jnp.stack([x_pad[:, s * F4:(s + 1) * F4] for s in range(SLICES)])

    deg_parts = _deg_kernel(row_p, col_p, ew_p)
    dis = _dis_tc(deg_parts).reshape(NPAD)

    p1, norm3 = _prop1(row3, col3, ew3, dis, x_split)
    p2 = _prop2(row3, col3, norm3, p1)

    return _final_tc(x, p1, p2, W, b)


# speedup vs baseline: 12.3820x; 1.0160x over previous
import dataclasses
import functools

import jax
import jax.numpy as jnp
from jax import lax
from jax.experimental import pallas as pl
from jax.experimental.pallas import tpu as pltpu
from jax.experimental.pallas import tpu_sc as plsc

N = 10000
E = 320000
F = 128
NPAD = 10240
LAMBDA_MAX = 2.0

NC = 2
NS = 16
NW = NC * NS
EPT = E // NW
B = 128
NB = (EPT + B - 1) // B
EPT_PAD = NB * B
F2 = F // NC
SLICES = 2 * NC
NSUB = SLICES // NC
F4 = F // SLICES
EPC = E // NS
NB2 = (EPC + B - 1) // B + ((EPC + B - 1) // B) % 2
EPC_PAD = NB2 * B
ROWS_PER_TILE = NPAD // NS

_mesh = plsc.VectorSubcoreMesh(core_axis_name="c", subcore_axis_name="s")

_sc_params = pltpu.CompilerParams()
if "needs_layout_passes" in pltpu.CompilerParams.__dataclass_fields__:
    _sc_params = dataclasses.replace(_sc_params, needs_layout_passes=False)
if "use_tc_tiling_on_sc" in pltpu.CompilerParams.__dataclass_fields__:
    _sc_params = dataclasses.replace(_sc_params, use_tc_tiling_on_sc=False)


def _wid():
    return lax.axis_index("s") * NC + lax.axis_index("c")


@functools.partial(
    pl.kernel,
    out_type=jax.ShapeDtypeStruct((NW, NPAD), jnp.float32),
    mesh=_mesh,
    compiler_params=_sc_params,
    scratch_types=[
        pltpu.VMEM((EPT_PAD,), jnp.int32),
        pltpu.VMEM((EPT_PAD,), jnp.int32),
        pltpu.VMEM((EPT_PAD,), jnp.float32),
        pltpu.VMEM((NPAD,), jnp.float32),
    ],
)
def _deg_kernel(row_hbm, col_hbm, ew_hbm, out_hbm, row_v, col_v, ew_v, deg_v):
    w = _wid()
    pltpu.sync_copy(row_hbm.at[w], row_v)
    pltpu.sync_copy(col_hbm.at[w], col_v)
    pltpu.sync_copy(ew_hbm.at[w], ew_v)

    zeros16 = jnp.zeros((16,), jnp.float32)

    @pl.loop(0, NPAD, step=16)
    def _(i):
        deg_v[pl.ds(i, 16)] = zeros16

    @pl.loop(0, EPT_PAD, step=16)
    def _(i):
        rv = row_v[pl.ds(i, 16)]
        cv = col_v[pl.ds(i, 16)]
        wv = ew_v[pl.ds(i, 16)]
        wm = jnp.where(rv != cv, wv, 0.0)
        plsc.addupdate_scatter(deg_v, [rv], wm)

    pltpu.sync_copy(deg_v, out_hbm.at[w])


def _dis_body(parts_ref, o_ref):
    deg = jnp.sum(parts_ref[...], axis=0, keepdims=True)
    safe = jnp.where(deg > 0, deg, 1.0)
    o_ref[...] = jnp.where(deg > 0, lax.rsqrt(safe), 0.0)


def _dis_tc(parts):
    return pl.pallas_call(
        _dis_body,
        out_shape=jax.ShapeDtypeStruct((1, NPAD), jnp.float32),
    )(parts)


def _prop_body(row_hbm, col_hbm, ew_hbm, dis_hbm, x_hbm,
               p1_hbm, p2_hbm,
               row_v, col_v, aux_v, dis_v, norm_v, g0, g1, s0, s1,
               acc_sh, sem0, sem1, ssem0, ssem1):
    core = lax.axis_index("c")
    sid = lax.axis_index("s")

    pltpu.sync_copy(row_hbm.at[sid], row_v)
    pltpu.sync_copy(col_hbm.at[sid], col_v)
    pltpu.sync_copy(ew_hbm.at[sid], aux_v)
    pltpu.sync_copy(dis_hbm, dis_v)
    scale = jnp.float32(2.0 / LAMBDA_MAX)

    @pl.loop(0, NB2)
    def _(b):
        @pl.loop(0, B, step=16)
        def _(j):
            ir = row_v[b, pl.ds(j, 16)]
            ic = col_v[b, pl.ds(j, 16)]
            dr = plsc.load_gather(dis_v, [ir])
            dc = plsc.load_gather(dis_v, [ic])
            ww = aux_v[b, pl.ds(j, 16)]
            nrm = -(dr * ww * dc) * scale
            norm_v[b, pl.ds(j, 16)] = jnp.where(ir != ic, nrm, 0.0)

    zeros16 = jnp.zeros((16,), jnp.float32)

    def scale_rows(gbuf, sbuf, b):
        @pl.loop(0, B, step=16)
        def _(j):
            nv = norm_v[b, pl.ds(j, 16)]
            for l in range(16):
                s = nv[l]
                for k in range(F4 // 16):
                    slc = pl.ds(k * 16, 16)
                    sbuf[j + l, slc] = gbuf[j + l, slc] * s

    def prop_pass(h_hbm, out_hbm):
        for sub in range(NSUB):
            sl = core * NSUB + sub

            @pl.loop(0, B)
            def _(j):
                for k in range(F4 // 16):
                    g0[j, pl.ds(k * 16, 16)] = zeros16

            for t in range(ROWS_PER_TILE // B):
                pltpu.sync_copy(g0, acc_sh.at[pl.ds(sid * ROWS_PER_TILE + t * B, B)])
            plsc.subcore_barrier()

            def gather(gbuf, sem, b):
                return pltpu.async_copy(h_hbm.at[sl].at[row_v.at[b]], gbuf, sem)

            def wait_gather(gbuf, sem, b):
                pltpu.make_async_copy(h_hbm.at[sl].at[row_v.at[b]], gbuf, sem).wait()

            def scatter(sbuf, sem, b):
                return pltpu.async_copy(sbuf, acc_sh.at[col_v.at[b]], sem, add=True)

            def wait_scatter(sbuf, sem, b):
                pltpu.make_async_copy(sbuf, acc_sh.at[col_v.at[b]], sem).wait()

            gather(g0, sem0, 0)
            gather(g1, sem1, 1)

            @pl.loop(0, NB2, step=2)
            def _(b):
                wait_gather(g0, sem0, b)

                @pl.when(b >= 2)
                def _():
                    wait_scatter(s0, ssem0, b - 2)

                scale_rows(g0, s0, b)
                scatter(s0, ssem0, b)

                @pl.when(b + 2 < NB2)
                def _():
                    gather(g0, sem0, b + 2)

                wait_gather(g1, sem1, b + 1)

                @pl.when(b >= 2)
                def _():
                    wait_scatter(s1, ssem1, b - 1)

                scale_rows(g1, s1, b + 1)
                scatter(s1, ssem1, b + 1)

                @pl.when(b + 3 < NB2)
                def _():
                    gather(g1, sem1, b + 3)

            wait_scatter(s0, ssem0, NB2 - 2)
            wait_scatter(s1, ssem1, NB2 - 1)

            plsc.subcore_barrier()
            pltpu.sync_copy(
                acc_sh.at[pl.ds(sid * ROWS_PER_TILE, ROWS_PER_TILE)],
                out_hbm.at[sl, pl.ds(sid * ROWS_PER_TILE, ROWS_PER_TILE)],
            )

    prop_pass(x_hbm, p1_hbm)
    prop_pass(p1_hbm, p2_hbm)


_ACC_TYPE = jax.ShapeDtypeStruct((SLICES, NPAD, F4), jnp.float32)

_prop12 = pl.kernel(
    _prop_body,
    out_type=(_ACC_TYPE, _ACC_TYPE),
    mesh=_mesh,
    compiler_params=_sc_params,
    scratch_types=[
        pltpu.VMEM((NB2, B), jnp.int32),
        pltpu.VMEM((NB2, B), jnp.int32),
        pltpu.VMEM((NB2, B), jnp.float32),
        pltpu.VMEM((NPAD,), jnp.float32),
        pltpu.VMEM((NB2, B), jnp.float32),
        pltpu.VMEM((B, F4), jnp.float32),
        pltpu.VMEM((B, F4), jnp.float32),
        pltpu.VMEM((B, F4), jnp.float32),
        pltpu.VMEM((B, F4), jnp.float32),
        pltpu.VMEM_SHARED((NPAD, F4), jnp.float32),
        pltpu.SemaphoreType.DMA,
        pltpu.SemaphoreType.DMA,
        pltpu.SemaphoreType.DMA,
        pltpu.SemaphoreType.DMA,
    ],
)


def _final_body(x_ref, p1_ref, p2_ref, w0_ref, w1_ref, w2_ref, b_ref, o_ref):
    dot = functools.partial(
        jnp.dot,
        preferred_element_type=jnp.float32,
        precision=lax.Precision.HIGHEST,
    )
    x = x_ref[...]
    y1 = jnp.concatenate([p1_ref[s] for s in range(SLICES)], axis=1)
    y2 = jnp.concatenate([p2_ref[s] for s in range(SLICES)], axis=1)
    t2 = 2.0 * y2 - x
    acc = dot(x, w0_ref[...]) + dot(y1, w1_ref[...]) + dot(t2, w2_ref[...])
    o_ref[...] = jnp.maximum(acc + b_ref[...][None, :], 0.0)


def _final_tc(x, p1, p2, W, b):
    blk = 2000
    return pl.pallas_call(
        _final_body,
        out_shape=jax.ShapeDtypeStruct((N, F), jnp.float32),
        grid=(N // blk,),
        in_specs=[
            pl.BlockSpec((blk, F), lambda i: (i, 0)),
            pl.BlockSpec((SLICES, blk, F4), lambda i: (0, i, 0)),
            pl.BlockSpec((SLICES, blk, F4), lambda i: (0, i, 0)),
            pl.BlockSpec((F, F), lambda i: (0, 0)),
            pl.BlockSpec((F, F), lambda i: (0, 0)),
            pl.BlockSpec((F, F), lambda i: (0, 0)),
            pl.BlockSpec((F,), lambda i: (0,)),
        ],
        out_specs=pl.BlockSpec((blk, F), lambda i: (i, 0)),
    )(x, p1, p2, W[0], W[1], W[2], b)


def kernel(x, edge_index, edge_weight, W, b):
    row = edge_index[0]
    col = edge_index[1]

    row_p = jnp.pad(row.reshape(NW, EPT), ((0, 0), (0, EPT_PAD - EPT)))
    col_p = jnp.pad(col.reshape(NW, EPT), ((0, 0), (0, EPT_PAD - EPT)))
    ew_p = jnp.pad(edge_weight.reshape(NW, EPT), ((0, 0), (0, EPT_PAD - EPT)))

    row3 = jnp.pad(row.reshape(NS, EPC), ((0, 0), (0, EPC_PAD - EPC))).reshape(NS, NB2, B)
    col3 = jnp.pad(col.reshape(NS, EPC), ((0, 0), (0, EPC_PAD - EPC))).reshape(NS, NB2, B)
    ew3 = jnp.pad(edge_weight.reshape(NS, EPC), ((0, 0), (0, EPC_PAD - EPC))).reshape(NS, NB2, B)

    x_pad = jnp.pad(x, ((0, NPAD - N), (0, 0)))
    x_split = jnp.stack([x_pad[:, s * F4:(s + 1) * F4] for s in range(SLICES)])

    deg_parts = _deg_kernel(row_p, col_p, ew_p)
    dis = _dis_tc(deg_parts).reshape(NPAD)

    p1, p2 = _prop12(row3, col3, ew3, dis, x_split)

    return _final_tc(x, p1, p2, W, b)
